# baseline (speedup 1.0000x reference)
import jax
import jax.numpy as jnp
from jax.experimental import pallas as pl
from jax.experimental.pallas import tpu as pltpu

_CB = 64
_CHUNK = 8


def _pool_kernel(x_ref, o_ref, scratch):
    _, cb, H, W = x_ref.shape
    Ho, Wo = H // 2, W // 2
    i = pl.program_id(0)
    par = jax.lax.rem(i, 2)
    prev = scratch[jax.lax.rem(i + 1, 2)]
    o_ref[...] = jnp.transpose(prev, (1, 2, 0))[None]
    for ci in range(cb // _CHUNK):
        xc = x_ref[0, ci * _CHUNK:(ci + 1) * _CHUNK]
        m2 = jnp.maximum(xc, jnp.roll(xc, -1, axis=1))
        v = m2.reshape(_CHUNK, H // 16, 2, 8, W)
        tile_even = v[:, :, 0]
        tile_odd = v[:, :, 1]
        s = jax.lax.broadcasted_iota(jnp.int32, tile_even.shape, 2)
        src = (2 * s) % 8
        g_even = jnp.take_along_axis(tile_even, src, axis=2)
        g_odd = jnp.take_along_axis(tile_odd, src, axis=2)
        y = jnp.where(s < 4, g_even, g_odd).reshape(_CHUNK, Ho, W)
        m1 = jnp.maximum(y, jnp.roll(y, -1, axis=2))
        lane = jax.lax.broadcasted_iota(jnp.int32, m1.shape, 2)
        out_c = jnp.take_along_axis(m1, (2 * lane) % W, axis=2)[:, :, :Wo]
        scratch[par, ci * _CHUNK:(ci + 1) * _CHUNK] = out_c


def kernel(x):
    B, C, H, W = x.shape
    nb = B * (C // _CB)
    grid = (nb + 1,)
    return jnp.transpose(
        pl.pallas_call(
            _pool_kernel,
            grid=grid,
            in_specs=[
                pl.BlockSpec(
                    (1, _CB, H, W),
                    lambda i: (jnp.minimum(i, nb - 1), 0, 0, 0),
                )
            ],
            out_specs=pl.BlockSpec(
                (1, H // 2, W // 2, _CB),
                lambda i: (jnp.maximum(i - 1, 0), 0, 0, 0),
            ),
            out_shape=jax.ShapeDtypeStruct((B, H // 2, W // 2, C), x.dtype),
            scratch_shapes=[pltpu.VMEM((2, _CB, H // 2, W // 2), x.dtype)],
            compiler_params=pltpu.CompilerParams(
                dimension_semantics=("arbitrary",),
            ),
        )(x),
        (0, 3, 1, 2),
    )
